# baseline (speedup 1.0000x reference)
import functools

import jax
import jax.numpy as jnp
from jax import lax
from jax.experimental import pallas as pl
from jax.experimental.pallas import tpu as pltpu
from jax.experimental.pallas import tpu_sc as plsc

N = 10000
NT = 10240
R = 208
E = 320000
EP = 327680
NW = 32
EPW = EP // NW
K = 64
NCH = EPW // K
RPW = NT // NW
D = 128

_mesh = plsc.VectorSubcoreMesh(core_axis_name="c", subcore_axis_name="s")


def _ent_body(x_ref, we, be, wh, bh, ws, bs, hp_ref, sm_ref):
    x = x_ref[...]
    eh = jnp.maximum(
        jnp.dot(x, we[...], preferred_element_type=jnp.float32) + be[...], 0.0)
    hp_ref[...] = jnp.dot(eh, wh[...], preferred_element_type=jnp.float32) + bh[...]
    sm_ref[...] = jnp.dot(eh, ws[...], preferred_element_type=jnp.float32) + bs[...]


def _tc_entity(ent_pad, W_ent, b_ent, W_head, b_head, W_self, b_self):
    blk = 1280
    grid = NT // blk
    mat = pl.BlockSpec((D, D), lambda i: (0, 0))
    vec = pl.BlockSpec((1, D), lambda i: (0, 0))
    row = pl.BlockSpec((blk, D), lambda i: (i, 0))
    return pl.pallas_call(
        _ent_body,
        grid=(grid,),
        in_specs=[row, mat, vec, mat, vec, mat, vec],
        out_specs=[row, row],
        out_shape=[jax.ShapeDtypeStruct((NT, D), jnp.float32)] * 2,
    )(ent_pad, W_ent, b_ent, W_head, b_head, W_self, b_self)


def _rel_body(rel_ref, wr, br, hq_row, hq_col, wq2e, bq2e, wq, be2e,
              relh_ref, smat_ref, qrow_ref):
    relh = jnp.dot(rel_ref[...], wr[...], preferred_element_type=jnp.float32) + br[...]
    relh_ref[...] = relh
    s = jax.nn.sigmoid(jnp.dot(relh, hq_col[...], preferred_element_type=jnp.float32))
    smat_ref[...] = jnp.broadcast_to(s, (R, D))
    qv = jnp.dot(hq_row[...], wq2e[...], preferred_element_type=jnp.float32) + bq2e[...]
    qrow_ref[...] = jnp.dot(qv, wq[...], preferred_element_type=jnp.float32) + be2e[...]


def _tc_rel(rel_pad, W_rel, b_rel, hq_row, hq_col, W_q2e, b_q2e, Wq, b_e2e):
    return pl.pallas_call(
        _rel_body,
        out_shape=[jax.ShapeDtypeStruct((R, D), jnp.float32),
                   jax.ShapeDtypeStruct((R, D), jnp.float32),
                   jax.ShapeDtypeStruct((1, D), jnp.float32)],
    )(rel_pad, W_rel, b_rel, hq_row, hq_col, W_q2e, b_q2e, Wq, b_e2e)


def _out_body(p0, p1, smr, wf, ws2, qrow, o_ref):
    f = p0[...] + p1[...]
    o_ref[...] = jnp.maximum(
        jnp.dot(f, wf[...], preferred_element_type=jnp.float32)
        + jnp.dot(smr[...], ws2[...], preferred_element_type=jnp.float32)
        + qrow[...], 0.0)


def _tc_out(p0, p1, sm, Wf, Ws, qrow):
    blk = 2000
    grid = N // blk
    mat = pl.BlockSpec((D, D), lambda i: (0, 0))
    vec = pl.BlockSpec((1, D), lambda i: (0, 0))
    row = pl.BlockSpec((blk, D), lambda i: (i, 0))
    return pl.pallas_call(
        _out_body,
        grid=(grid,),
        in_specs=[row, row, row, mat, mat, vec],
        out_specs=row,
        out_shape=jax.ShapeDtypeStruct((N, D), jnp.float32),
    )(p0, p1, sm, Wf, Ws, qrow)


@functools.partial(
    pl.kernel,
    out_type=(jax.ShapeDtypeStruct((NT, D), jnp.float32),
              jax.ShapeDtypeStruct((NT, D), jnp.float32),
              jax.ShapeDtypeStruct((NT, D), jnp.float32),
              jax.ShapeDtypeStruct((NT, D), jnp.float32),
              jax.ShapeDtypeStruct((NT, D), jnp.float32)),
    mesh=_mesh,
    scratch_types=(
        pltpu.VMEM((R, D), jnp.float32),
        pltpu.VMEM((R * 16,), jnp.float32),
        pltpu.VMEM((3 * K // 2,), jnp.int32),
        pltpu.VMEM((3 * K // 2,), jnp.int32),
        pltpu.VMEM((K,), jnp.int32),
        pltpu.VMEM((K,), jnp.int32),
        pltpu.VMEM((K,), jnp.int32),
        pltpu.VMEM((K,), jnp.int32),
        pltpu.VMEM((K,), jnp.int32),
        pltpu.VMEM((K,), jnp.int32),
        pltpu.VMEM((K, D), jnp.float32),
        pltpu.VMEM((K, D), jnp.float32),
        pltpu.VMEM_SHARED((NT, D), jnp.float32),
        pltpu.SemaphoreType.DMA,
        pltpu.SemaphoreType.DMA,
    ),
)
def _sc_edges(le_hbm, hp_tbl, sm_tbl, idx_hbm, relh_hbm, sbf_hbm,
              f2e0, f2e1, hp0_hbm, hp1_hbm, sm_out,
              relh_v, sbf_v, pk0, pk1, src0, src1, rel0, rel1,
              dst0, dst1, rows0, rows1, f2e_sp, sem0, sem1):
    c = lax.axis_index("c")
    s = lax.axis_index("s")
    wid = s * 2 + c
    stripe = NT // 16
    soff = s * stripe

    pltpu.sync_copy(relh_hbm, relh_v)
    pltpu.sync_copy(sbf_hbm, sbf_v)

    bufs = ((pk0, src0, rel0, dst0, rows0, sem0),
            (pk1, src1, rel1, dst1, rows1, sem1))

    def hp_it(i, carry):
        base = pl.multiple_of(s * stripe + i * K, 8)
        pltpu.sync_copy(le_hbm.at[pl.ds(base, K)], src0)
        pltpu.async_copy(hp_tbl.at[src0], rows0, sem0).wait()

        @pl.when(c == 0)
        def _():
            pltpu.sync_copy(rows0, hp0_hbm.at[pl.ds(base, K)])

        @pl.when(c == 1)
        def _():
            pltpu.sync_copy(rows0, hp1_hbm.at[pl.ds(base, K)])

        return carry

    lax.fori_loop(0, stripe // K, hp_it, 0)

    def sm_it(i, carry):
        base = pl.multiple_of(wid * RPW + i * K, 8)
        pltpu.sync_copy(le_hbm.at[pl.ds(base, K)], src0)
        pltpu.async_copy(sm_tbl.at[src0], rows0, sem0).wait()
        pltpu.sync_copy(rows0, sm_out.at[pl.ds(base, K)])
        return carry

    lax.fori_loop(0, RPW // K, sm_it, 0)

    zv = jnp.zeros((16,), jnp.float32)

    def zrow(e, carry):
        for j in range(D // 16):
            rows0[e, pl.ds(16 * j, 16)] = zv
        return carry

    lax.fori_loop(0, K, zrow, 0)
    for q in range(stripe // K):
        pltpu.sync_copy(rows0, f2e_sp.at[pl.ds(soff + K * q, K)])
    plsc.subcore_barrier()

    def fetch(ci, b):
        pk_v, src_v, rel_v, dst_v, rows_v, sem = bufs[b]
        base = pl.multiple_of((wid * NCH + ci) * (3 * K // 2), 8)
        pltpu.sync_copy(idx_hbm.at[pl.ds(base, 3 * K // 2)], pk_v)
        for t, arr_v in enumerate((src_v, rel_v, dst_v)):
            for g in range(K // 32):
                w = pk_v[pl.ds(t * (K // 2) + 16 * g, 16)]
                arr_v[pl.ds(32 * g, 16)] = w & 0xFFFF
                arr_v[pl.ds(32 * g + 16, 16)] = lax.shift_right_logical(w, 16)

        @pl.when(c == 0)
        def _():
            pltpu.async_copy(hp0_hbm.at[src_v], rows_v, sem)

        @pl.when(c == 1)
        def _():
            pltpu.async_copy(hp1_hbm.at[src_v], rows_v, sem)

    def process(ci, b):
        pk_v, src_v, rel_v, dst_v, rows_v, sem = bufs[b]

        @pl.when(ci + 1 < NCH)
        def _():
            fetch(ci + 1, 1 - b)

        pltpu.make_async_copy(hp0_hbm.at[src_v], rows_v, sem).wait()

        def grp(g, carry2):
            rel16 = rel_v[pl.ds(g * 16, 16)]
            for l in range(16):
                rel_e = rel16[l]
                e = g * 16 + l
                sv = sbf_v[pl.ds(rel_e * 16, 16)]
                rs = [rows_v[e, pl.ds(16 * j, 16)] for j in range(D // 16)]
                rhs = [relh_v[rel_e, pl.ds(16 * j, 16)] for j in range(D // 16)]
                for j in range(D // 16):
                    rows_v[e, pl.ds(16 * j, 16)] = sv * jnp.maximum(rs[j] + rhs[j], 0.0)
            return carry2

        lax.fori_loop(0, K // 16, grp, 0)
        pltpu.sync_copy(rows_v, f2e_sp.at[dst_v], add=True)

    fetch(0, 0)

    def pair(ci2, carry):
        process(2 * ci2, 0)
        process(2 * ci2 + 1, 1)
        return carry

    lax.fori_loop(0, NCH // 2, pair, 0)
    plsc.subcore_barrier()

    @pl.when(c == 0)
    def _():
        pltpu.sync_copy(f2e_sp.at[pl.ds(soff, stripe)],
                        f2e0.at[pl.ds(soff, stripe)])

    @pl.when(c == 1)
    def _():
        pltpu.sync_copy(f2e_sp.at[pl.ds(soff, stripe)],
                        f2e1.at[pl.ds(soff, stripe)])


def kernel(local_entity, edge_index, edge_rel, h_q, entity_table, relation_table,
           W_ent, b_ent, W_rel, b_rel, W_head, b_head, W_self, b_self,
           W_q2e, b_q2e, W_e2e, b_e2e):
    f32 = jnp.float32
    ent_pad = jnp.pad(entity_table.astype(f32), ((0, NT - entity_table.shape[0]), (0, 0)))
    rel_pad = jnp.pad(relation_table.astype(f32), ((0, R - relation_table.shape[0]), (0, 0)))
    le_pad = jnp.pad(local_entity.astype(jnp.int32), (0, NT - N))

    def pack(x):
        x32 = x.reshape(-1, 2, 16)
        return (x32[:, 0, :] | (x32[:, 1, :] << 16)).reshape(-1)

    src_p = pack(jnp.pad(edge_index[0].astype(jnp.int32), (0, EP - E)))
    rel_p = pack(jnp.pad(edge_rel.astype(jnp.int32), (0, EP - E)))
    trash = N + jnp.arange(EP - E, dtype=jnp.int32) % (NT - N)
    dst_p = pack(jnp.concatenate([edge_index[1].astype(jnp.int32), trash]))
    idx_p = jnp.concatenate([src_p.reshape(-1, K // 2),
                             rel_p.reshape(-1, K // 2),
                             dst_p.reshape(-1, K // 2)], axis=1).reshape(-1)

    def row(b):
        return b.reshape(1, D).astype(f32)

    Wq = W_e2e[0:D]
    Wf = W_e2e[D:2 * D]
    Ws = W_e2e[2 * D:3 * D]
    hq_row = h_q.reshape(1, D).astype(f32)
    hq_col = h_q.reshape(D, 1).astype(f32)

    hp_tbl, sm_tbl = _tc_entity(ent_pad, W_ent, row(b_ent), W_head, row(b_head),
                                W_self, row(b_self))
    relh, smat, qrow = _tc_rel(rel_pad, W_rel, row(b_rel), hq_row, hq_col,
                               W_q2e, row(b_q2e), Wq, row(b_e2e))
    sbf = smat[:, :16].reshape(R * 16)
    f2e0, f2e1, _hp0, _hp1, sm_g = _sc_edges(le_pad, hp_tbl, sm_tbl,
                                             idx_p, relh, sbf)
    return _tc_out(f2e0[:N], f2e1[:N], sm_g[:N], Wf, Ws, qrow)


# speedup vs baseline: 1.3204x; 1.3204x over previous
import functools

import jax
import jax.numpy as jnp
from jax import lax
from jax.experimental import pallas as pl
from jax.experimental.pallas import tpu as pltpu
from jax.experimental.pallas import tpu_sc as plsc

N = 10000
NT = 10240
R = 208
E = 320000
EP = 327680
NW = 32
EPW = EP // NW
K = 64
NCH = EPW // K
RPW = NT // NW
D = 128

_mesh = plsc.VectorSubcoreMesh(core_axis_name="c", subcore_axis_name="s")


def _ent_body(x_ref, we, be, wh, bh, ws, bs, hp_ref, sm_ref):
    x = x_ref[...]
    eh = jnp.maximum(
        jnp.dot(x, we[...], preferred_element_type=jnp.float32) + be[...], 0.0)
    hp_ref[...] = jnp.dot(eh, wh[...], preferred_element_type=jnp.float32) + bh[...]
    sm_ref[...] = jnp.dot(eh, ws[...], preferred_element_type=jnp.float32) + bs[...]


def _tc_entity(ent_pad, W_ent, b_ent, W_head, b_head, W_self, b_self):
    blk = 1280
    grid = NT // blk
    mat = pl.BlockSpec((D, D), lambda i: (0, 0))
    vec = pl.BlockSpec((1, D), lambda i: (0, 0))
    row = pl.BlockSpec((blk, D), lambda i: (i, 0))
    return pl.pallas_call(
        _ent_body,
        grid=(grid,),
        in_specs=[row, mat, vec, mat, vec, mat, vec],
        out_specs=[row, row],
        out_shape=[jax.ShapeDtypeStruct((NT, D), jnp.float32)] * 2,
    )(ent_pad, W_ent, b_ent, W_head, b_head, W_self, b_self)


def _rel_body(rel_ref, wr, br, hq_row, hq_col, wq2e, bq2e, wq, be2e,
              relh_ref, smat_ref, qrow_ref):
    relh = jnp.dot(rel_ref[...], wr[...], preferred_element_type=jnp.float32) + br[...]
    relh_ref[...] = relh
    s = jax.nn.sigmoid(jnp.dot(relh, hq_col[...], preferred_element_type=jnp.float32))
    smat_ref[...] = jnp.broadcast_to(s, (R, D))
    qv = jnp.dot(hq_row[...], wq2e[...], preferred_element_type=jnp.float32) + bq2e[...]
    qrow_ref[...] = jnp.dot(qv, wq[...], preferred_element_type=jnp.float32) + be2e[...]


def _tc_rel(rel_pad, W_rel, b_rel, hq_row, hq_col, W_q2e, b_q2e, Wq, b_e2e):
    return pl.pallas_call(
        _rel_body,
        out_shape=[jax.ShapeDtypeStruct((R, D), jnp.float32),
                   jax.ShapeDtypeStruct((R, D), jnp.float32),
                   jax.ShapeDtypeStruct((1, D), jnp.float32)],
    )(rel_pad, W_rel, b_rel, hq_row, hq_col, W_q2e, b_q2e, Wq, b_e2e)


def _out_body(p0, p1, smr, wf, ws2, qrow, o_ref):
    f = p0[...] + p1[...]
    o_ref[...] = jnp.maximum(
        jnp.dot(f, wf[...], preferred_element_type=jnp.float32)
        + jnp.dot(smr[...], ws2[...], preferred_element_type=jnp.float32)
        + qrow[...], 0.0)


def _tc_out(p0, p1, sm, Wf, Ws, qrow):
    blk = 2000
    grid = N // blk
    mat = pl.BlockSpec((D, D), lambda i: (0, 0))
    vec = pl.BlockSpec((1, D), lambda i: (0, 0))
    row = pl.BlockSpec((blk, D), lambda i: (i, 0))
    return pl.pallas_call(
        _out_body,
        grid=(grid,),
        in_specs=[row, row, row, mat, mat, vec],
        out_specs=row,
        out_shape=jax.ShapeDtypeStruct((N, D), jnp.float32),
    )(p0, p1, sm, Wf, Ws, qrow)


@functools.partial(
    pl.kernel,
    out_type=(jax.ShapeDtypeStruct((NT, D), jnp.float32),
              jax.ShapeDtypeStruct((NT, D), jnp.float32)),
    mesh=_mesh,
    scratch_types=(
        pltpu.VMEM((K,), jnp.int32),
        pltpu.VMEM((K, D), jnp.float32),
        pltpu.SemaphoreType.DMA,
    ),
)
def _sc_gather(le_hbm, hp_tbl, sm_tbl, hp_out, sm_out, le_v, buf, sem):
    wid = lax.axis_index("s") * 2 + lax.axis_index("c")

    def it(i, carry):
        base = pl.multiple_of(wid * RPW + i * K, 8)
        pltpu.sync_copy(le_hbm.at[pl.ds(base, K)], le_v)
        pltpu.async_copy(hp_tbl.at[le_v], buf, sem).wait()
        pltpu.sync_copy(buf, hp_out.at[pl.ds(base, K)])
        pltpu.async_copy(sm_tbl.at[le_v], buf, sem).wait()
        pltpu.sync_copy(buf, sm_out.at[pl.ds(base, K)])
        return carry

    lax.fori_loop(0, RPW // K, it, 0)


@functools.partial(
    pl.kernel,
    out_type=(jax.ShapeDtypeStruct((NT, D), jnp.float32),
              jax.ShapeDtypeStruct((NT, D), jnp.float32)),
    mesh=_mesh,
    scratch_types=(
        pltpu.VMEM((R, D), jnp.float32),
        pltpu.VMEM((R * 16,), jnp.float32),
        pltpu.VMEM((3 * K,), jnp.int32),
        pltpu.VMEM((3 * K,), jnp.int32),
        pltpu.VMEM((K,), jnp.int32),
        pltpu.VMEM((K,), jnp.int32),
        pltpu.VMEM((K, D), jnp.float32),
        pltpu.VMEM((K, D), jnp.float32),
        pltpu.VMEM_SHARED((NT, D), jnp.float32),
        pltpu.SemaphoreType.DMA,
        pltpu.SemaphoreType.DMA,
        pltpu.SemaphoreType.DMA,
        pltpu.SemaphoreType.DMA,
    ),
)
def _sc_edges(idx_hbm, hp_hbm, relh_hbm, sbf_hbm,
              f2e0, f2e1, relh_v, sbf_v, pk0, pk1, dst0, dst1, rows0, rows1,
              f2e_sp, sem0, sem1, ssem0, ssem1):
    c = lax.axis_index("c")
    s = lax.axis_index("s")
    wid = s * 2 + c
    stripe = NT // 16
    soff = s * stripe

    pltpu.sync_copy(relh_hbm, relh_v)
    pltpu.sync_copy(sbf_hbm, sbf_v)

    bufs = ((pk0, dst0, rows0, sem0, ssem0),
            (pk1, dst1, rows1, sem1, ssem1))

    zv = jnp.zeros((16,), jnp.float32)

    def zrow(e, carry):
        for j in range(D // 16):
            rows0[e, pl.ds(16 * j, 16)] = zv
        return carry

    lax.fori_loop(0, K, zrow, 0)
    for q in range(stripe // K):
        pltpu.sync_copy(rows0, f2e_sp.at[pl.ds(soff + K * q, K)])
    plsc.subcore_barrier()

    def fetch(ci, b):
        pk_v, dst_v, rows_v, sem, ssem = bufs[b]

        @pl.when(ci >= 2)
        def _():
            pltpu.make_async_copy(rows_v, f2e_sp.at[dst_v], ssem).wait()

        base = pl.multiple_of((wid * NCH + ci) * (3 * K), 8)
        pltpu.sync_copy(idx_hbm.at[pl.ds(base, 3 * K)], pk_v)
        for g in range(K // 16):
            dst_v[pl.ds(16 * g, 16)] = pk_v[pl.ds(2 * K + 16 * g, 16)]
        pltpu.async_copy(hp_hbm.at[pk_v.at[pl.ds(0, K)]], rows_v, sem)

    def process(ci, b):
        pk_v, dst_v, rows_v, sem, ssem = bufs[b]

        @pl.when(ci + 1 < NCH)
        def _():
            fetch(ci + 1, 1 - b)

        pltpu.make_async_copy(hp_hbm.at[pk_v.at[pl.ds(0, K)]], rows_v, sem).wait()

        def grp(g, carry2):
            rel16 = pk_v[pl.ds(K + g * 16, 16)]
            for l in range(16):
                rel_e = rel16[l]
                e = g * 16 + l
                sv = sbf_v[pl.ds(rel_e * 16, 16)]
                rs = [rows_v[e, pl.ds(16 * j, 16)] for j in range(D // 16)]
                rhs = [relh_v[rel_e, pl.ds(16 * j, 16)] for j in range(D // 16)]
                for j in range(D // 16):
                    rows_v[e, pl.ds(16 * j, 16)] = sv * jnp.maximum(rs[j] + rhs[j], 0.0)
            return carry2

        lax.fori_loop(0, K // 16, grp, 0)
        pltpu.async_copy(rows_v, f2e_sp.at[dst_v], ssem, add=True)

    fetch(0, 0)

    def pair(ci2, carry):
        process(2 * ci2, 0)
        process(2 * ci2 + 1, 1)
        return carry

    lax.fori_loop(0, NCH // 2, pair, 0)
    pltpu.make_async_copy(rows0, f2e_sp.at[dst0], ssem0).wait()
    pltpu.make_async_copy(rows1, f2e_sp.at[dst1], ssem1).wait()
    plsc.subcore_barrier()

    @pl.when(c == 0)
    def _():
        pltpu.sync_copy(f2e_sp.at[pl.ds(soff, stripe)],
                        f2e0.at[pl.ds(soff, stripe)])

    @pl.when(c == 1)
    def _():
        pltpu.sync_copy(f2e_sp.at[pl.ds(soff, stripe)],
                        f2e1.at[pl.ds(soff, stripe)])


def kernel(local_entity, edge_index, edge_rel, h_q, entity_table, relation_table,
           W_ent, b_ent, W_rel, b_rel, W_head, b_head, W_self, b_self,
           W_q2e, b_q2e, W_e2e, b_e2e):
    f32 = jnp.float32
    ent_pad = jnp.pad(entity_table.astype(f32), ((0, NT - entity_table.shape[0]), (0, 0)))
    rel_pad = jnp.pad(relation_table.astype(f32), ((0, R - relation_table.shape[0]), (0, 0)))
    le_pad = jnp.pad(local_entity.astype(jnp.int32), (0, NT - N))

    src_p = jnp.pad(edge_index[0].astype(jnp.int32), (0, EP - E))
    rel_p = jnp.pad(edge_rel.astype(jnp.int32), (0, EP - E))
    trash = N + jnp.arange(EP - E, dtype=jnp.int32) % (NT - N)
    dst_p = jnp.concatenate([edge_index[1].astype(jnp.int32), trash])
    idx_p = jnp.concatenate([src_p.reshape(-1, 1, K), rel_p.reshape(-1, 1, K),
                             dst_p.reshape(-1, 1, K)], axis=1).reshape(-1)

    def row(b):
        return b.reshape(1, D).astype(f32)

    Wq = W_e2e[0:D]
    Wf = W_e2e[D:2 * D]
    Ws = W_e2e[2 * D:3 * D]
    hq_row = h_q.reshape(1, D).astype(f32)
    hq_col = h_q.reshape(D, 1).astype(f32)

    hp_tbl, sm_tbl = _tc_entity(ent_pad, W_ent, row(b_ent), W_head, row(b_head),
                                W_self, row(b_self))
    relh, smat, qrow = _tc_rel(rel_pad, W_rel, row(b_rel), hq_row, hq_col,
                               W_q2e, row(b_q2e), Wq, row(b_e2e))
    sbf = smat[:, :16].reshape(R * 16)
    hp_g, sm_g = _sc_gather(le_pad, hp_tbl, sm_tbl)
    f2e0, f2e1 = _sc_edges(idx_p, hp_g, relh, sbf)
    return _tc_out(f2e0[:N], f2e1[:N], sm_g[:N], Wf, Ws, qrow)


# speedup vs baseline: 1.4685x; 1.1122x over previous
import functools

import jax
import jax.numpy as jnp
from jax import lax
from jax.experimental import pallas as pl
from jax.experimental.pallas import tpu as pltpu
from jax.experimental.pallas import tpu_sc as plsc

N = 10000
NT = 10240
R = 208
E = 320000
EP = 327680
NW = 32
EPW = EP // NW
K = 64
NCH = EPW // K
RPW = NT // NW
D = 128

_mesh = plsc.VectorSubcoreMesh(core_axis_name="c", subcore_axis_name="s")


def _ent_body(x_ref, we, be, wh, bh, ws, bs, hp_ref, sm_ref):
    x = x_ref[...]
    eh = jnp.maximum(
        jnp.dot(x, we[...], preferred_element_type=jnp.float32) + be[...], 0.0)
    hp_ref[...] = jnp.dot(eh, wh[...], preferred_element_type=jnp.float32) + bh[...]
    sm_ref[...] = jnp.dot(eh, ws[...], preferred_element_type=jnp.float32) + bs[...]


def _tc_entity(ent_pad, W_ent, b_ent, W_head, b_head, W_self, b_self):
    blk = 1280
    grid = NT // blk
    mat = pl.BlockSpec((D, D), lambda i: (0, 0))
    vec = pl.BlockSpec((1, D), lambda i: (0, 0))
    row = pl.BlockSpec((blk, D), lambda i: (i, 0))
    return pl.pallas_call(
        _ent_body,
        grid=(grid,),
        in_specs=[row, mat, vec, mat, vec, mat, vec],
        out_specs=[row, row],
        out_shape=[jax.ShapeDtypeStruct((NT, D), jnp.float32)] * 2,
    )(ent_pad, W_ent, b_ent, W_head, b_head, W_self, b_self)


def _rel_body(rel_ref, wr, br, hq_row, hq_col, wq2e, bq2e, wq, be2e,
              relh_ref, smat_ref, qrow_ref):
    relh = jnp.dot(rel_ref[...], wr[...], preferred_element_type=jnp.float32) + br[...]
    relh_ref[...] = relh
    s = jax.nn.sigmoid(jnp.dot(relh, hq_col[...], preferred_element_type=jnp.float32))
    smat_ref[...] = jnp.broadcast_to(s, (R, D))
    qv = jnp.dot(hq_row[...], wq2e[...], preferred_element_type=jnp.float32) + bq2e[...]
    qrow_ref[...] = jnp.dot(qv, wq[...], preferred_element_type=jnp.float32) + be2e[...]


def _tc_rel(rel_pad, W_rel, b_rel, hq_row, hq_col, W_q2e, b_q2e, Wq, b_e2e):
    return pl.pallas_call(
        _rel_body,
        out_shape=[jax.ShapeDtypeStruct((R, D), jnp.float32),
                   jax.ShapeDtypeStruct((R, D), jnp.float32),
                   jax.ShapeDtypeStruct((1, D), jnp.float32)],
    )(rel_pad, W_rel, b_rel, hq_row, hq_col, W_q2e, b_q2e, Wq, b_e2e)


def _out_body(p0, p1, smr, wf, ws2, qrow, o_ref):
    f = p0[...] + p1[...]
    o_ref[...] = jnp.maximum(
        jnp.dot(f, wf[...], preferred_element_type=jnp.float32)
        + jnp.dot(smr[...], ws2[...], preferred_element_type=jnp.float32)
        + qrow[...], 0.0)


def _tc_out(p0, p1, sm, Wf, Ws, qrow):
    blk = 2000
    grid = N // blk
    mat = pl.BlockSpec((D, D), lambda i: (0, 0))
    vec = pl.BlockSpec((1, D), lambda i: (0, 0))
    row = pl.BlockSpec((blk, D), lambda i: (i, 0))
    return pl.pallas_call(
        _out_body,
        grid=(grid,),
        in_specs=[row, row, row, mat, mat, vec],
        out_specs=row,
        out_shape=jax.ShapeDtypeStruct((N, D), jnp.float32),
    )(p0, p1, sm, Wf, Ws, qrow)


@functools.partial(
    pl.kernel,
    out_type=(jax.ShapeDtypeStruct((NT, D), jnp.float32),
              jax.ShapeDtypeStruct((NT, D), jnp.float32)),
    mesh=_mesh,
    scratch_types=(
        pltpu.VMEM((K,), jnp.int32),
        pltpu.VMEM((K, D), jnp.float32),
        pltpu.SemaphoreType.DMA,
    ),
)
def _sc_gather(le_hbm, hp_tbl, sm_tbl, hp_out, sm_out, le_v, buf, sem):
    c = lax.axis_index("c")
    s = lax.axis_index("s")
    row0 = jnp.where(c == 0, s * 384, 16 * 384 + s * 256)
    iters = jnp.where(c == 0, 384 // K, 256 // K)

    def it(i, carry):
        base = pl.multiple_of(row0 + i * K, 8)
        pltpu.sync_copy(le_hbm.at[pl.ds(base, K)], le_v)
        pltpu.async_copy(hp_tbl.at[le_v], buf, sem).wait()
        pltpu.sync_copy(buf, hp_out.at[pl.ds(base, K)])
        pltpu.async_copy(sm_tbl.at[le_v], buf, sem).wait()
        pltpu.sync_copy(buf, sm_out.at[pl.ds(base, K)])
        return carry

    lax.fori_loop(0, iters, it, 0)


@functools.partial(
    pl.kernel,
    out_type=(jax.ShapeDtypeStruct((NT, D), jnp.float32),
              jax.ShapeDtypeStruct((NT, D), jnp.float32)),
    mesh=_mesh,
    scratch_types=(
        pltpu.VMEM((R, D), jnp.float32),
        pltpu.VMEM((R * 16,), jnp.float32),
        pltpu.VMEM((3 * K,), jnp.int32),
        pltpu.VMEM((3 * K,), jnp.int32),
        pltpu.VMEM((K,), jnp.int32),
        pltpu.VMEM((K,), jnp.int32),
        pltpu.VMEM((K, D), jnp.float32),
        pltpu.VMEM((K, D), jnp.float32),
        pltpu.VMEM_SHARED((NT, D), jnp.float32),
        pltpu.SemaphoreType.DMA,
        pltpu.SemaphoreType.DMA,
        pltpu.SemaphoreType.DMA,
        pltpu.SemaphoreType.DMA,
    ),
)
def _sc_edges(idx_hbm, hp_hbm, relh_hbm, sbf_hbm,
              f2e0, f2e1, relh_v, sbf_v, pk0, pk1, dst0, dst1, rows0, rows1,
              f2e_sp, sem0, sem1, ssem0, ssem1):
    c = lax.axis_index("c")
    s = lax.axis_index("s")
    stripe = NT // 16
    soff = s * stripe
    NCH0, NCH1 = 204, 116
    chunk0 = jnp.where(c == 0, s * NCH0, 16 * NCH0 + s * NCH1)
    nch = jnp.where(c == 0, NCH0, NCH1)

    pltpu.sync_copy(relh_hbm, relh_v)
    pltpu.sync_copy(sbf_hbm, sbf_v)

    bufs = ((pk0, dst0, rows0, sem0, ssem0),
            (pk1, dst1, rows1, sem1, ssem1))

    zv = jnp.zeros((16,), jnp.float32)

    def zrow(e, carry):
        for j in range(D // 16):
            rows0[e, pl.ds(16 * j, 16)] = zv
        return carry

    lax.fori_loop(0, K, zrow, 0)
    for q in range(stripe // K):
        pltpu.sync_copy(rows0, f2e_sp.at[pl.ds(soff + K * q, K)])
    plsc.subcore_barrier()

    def fetch(ci, b):
        pk_v, dst_v, rows_v, sem, ssem = bufs[b]

        @pl.when(ci >= 2)
        def _():
            pltpu.make_async_copy(rows_v, f2e_sp.at[dst_v], ssem).wait()

        base = pl.multiple_of((chunk0 + ci) * (3 * K), 8)
        pltpu.sync_copy(idx_hbm.at[pl.ds(base, 3 * K)], pk_v)
        for g in range(K // 16):
            dst_v[pl.ds(16 * g, 16)] = pk_v[pl.ds(2 * K + 16 * g, 16)]
        pltpu.async_copy(hp_hbm.at[pk_v.at[pl.ds(0, K)]], rows_v, sem)

    def process(ci, b):
        pk_v, dst_v, rows_v, sem, ssem = bufs[b]

        @pl.when(ci + 1 < nch)
        def _():
            fetch(ci + 1, 1 - b)

        pltpu.make_async_copy(hp_hbm.at[pk_v.at[pl.ds(0, K)]], rows_v, sem).wait()

        def grp(g, carry2):
            rel16 = pk_v[pl.ds(K + g * 16, 16)]
            for l in range(16):
                rel_e = rel16[l]
                e = g * 16 + l
                sv = sbf_v[pl.ds(rel_e * 16, 16)]
                rs = [rows_v[e, pl.ds(16 * j, 16)] for j in range(D // 16)]
                rhs = [relh_v[rel_e, pl.ds(16 * j, 16)] for j in range(D // 16)]
                for j in range(D // 16):
                    rows_v[e, pl.ds(16 * j, 16)] = sv * jnp.maximum(rs[j] + rhs[j], 0.0)
            return carry2

        lax.fori_loop(0, K // 16, grp, 0)
        pltpu.async_copy(rows_v, f2e_sp.at[dst_v], ssem, add=True)

    fetch(0, 0)

    def pair(ci2, carry):
        process(2 * ci2, 0)
        process(2 * ci2 + 1, 1)
        return carry

    lax.fori_loop(0, nch // 2, pair, 0)
    pltpu.make_async_copy(rows0, f2e_sp.at[dst0], ssem0).wait()
    pltpu.make_async_copy(rows1, f2e_sp.at[dst1], ssem1).wait()
    plsc.subcore_barrier()

    @pl.when(c == 0)
    def _():
        pltpu.sync_copy(f2e_sp.at[pl.ds(soff, stripe)],
                        f2e0.at[pl.ds(soff, stripe)])

    @pl.when(c == 1)
    def _():
        pltpu.sync_copy(f2e_sp.at[pl.ds(soff, stripe)],
                        f2e1.at[pl.ds(soff, stripe)])


def kernel(local_entity, edge_index, edge_rel, h_q, entity_table, relation_table,
           W_ent, b_ent, W_rel, b_rel, W_head, b_head, W_self, b_self,
           W_q2e, b_q2e, W_e2e, b_e2e):
    f32 = jnp.float32
    ent_pad = jnp.pad(entity_table.astype(f32), ((0, NT - entity_table.shape[0]), (0, 0)))
    rel_pad = jnp.pad(relation_table.astype(f32), ((0, R - relation_table.shape[0]), (0, 0)))
    le_pad = jnp.pad(local_entity.astype(jnp.int32), (0, NT - N))

    src_p = jnp.pad(edge_index[0].astype(jnp.int32), (0, EP - E))
    rel_p = jnp.pad(edge_rel.astype(jnp.int32), (0, EP - E))
    trash = N + jnp.arange(EP - E, dtype=jnp.int32) % (NT - N)
    dst_p = jnp.concatenate([edge_index[1].astype(jnp.int32), trash])
    idx_p = jnp.concatenate([src_p.reshape(-1, 1, K), rel_p.reshape(-1, 1, K),
                             dst_p.reshape(-1, 1, K)], axis=1).reshape(-1)

    def row(b):
        return b.reshape(1, D).astype(f32)

    Wq = W_e2e[0:D]
    Wf = W_e2e[D:2 * D]
    Ws = W_e2e[2 * D:3 * D]
    hq_row = h_q.reshape(1, D).astype(f32)
    hq_col = h_q.reshape(D, 1).astype(f32)

    hp_tbl, sm_tbl = _tc_entity(ent_pad, W_ent, row(b_ent), W_head, row(b_head),
                                W_self, row(b_self))
    relh, smat, qrow = _tc_rel(rel_pad, W_rel, row(b_rel), hq_row, hq_col,
                               W_q2e, row(b_q2e), Wq, row(b_e2e))
    sbf = smat[:, :16].reshape(R * 16)
    hp_g, sm_g = _sc_gather(le_pad, hp_tbl, sm_tbl)
    f2e0, f2e1 = _sc_edges(idx_p, hp_g, relh, sbf)
    return _tc_out(f2e0[:N], f2e1[:N], sm_g[:N], Wf, Ws, qrow)
